# baseline (speedup 1.0000x reference)
import functools

import jax
import jax.numpy as jnp
from jax import lax
from jax.experimental import pallas as pl
from jax.experimental.pallas import tpu as pltpu
from jax.experimental.pallas import tpu_sc as plsc

MAXLEN = 200
DIM = 64
LANES = 16
NC, NS = 2, 16
NW = NC * NS
NBUF = 2
ROUNDS = MAXLEN // NBUF
SUB = 8


def _pe_table():
    position = jnp.arange(MAXLEN, dtype=jnp.float32)[:, None]
    div_term = jnp.exp(
        jnp.arange(0, DIM, 2, dtype=jnp.float32) * (-jnp.log(10000.0) / DIM)
    )
    pe = jnp.zeros((MAXLEN, (DIM + 1) // 2 * 2), dtype=jnp.float32)
    pe = pe.at[:, 0::2].set(jnp.sin(position * div_term))
    pe = pe.at[:, 1::2].set(jnp.cos(position * div_term))
    return pe[:, :DIM]


def _make_sc_call(batch):
    bblk = batch // NW
    tt_n = MAXLEN // SUB
    gblk = bblk // LANES
    mesh = plsc.VectorSubcoreMesh(core_axis_name="c", subcore_axis_name="s")

    @functools.partial(
        pl.kernel,
        out_type=jax.ShapeDtypeStruct(
            (MAXLEN, DIM // SUB, NW, SUB, bblk), jnp.float32
        ),
        mesh=mesh,
        scratch_types=[
            pltpu.VMEM((tt_n, SUB, bblk), jnp.int32),
            pltpu.VMEM((MAXLEN * DIM,), jnp.float32),
            [pltpu.VMEM((bblk, 2 * DIM), jnp.float32) for _ in range(NBUF)],
            [pltpu.VMEM((DIM // SUB, SUB, bblk), jnp.float32) for _ in range(NBUF)],
            [pltpu.VMEM((bblk,), jnp.int32) for _ in range(NBUF)],
            [pltpu.VMEM((bblk,), jnp.int32) for _ in range(NBUF)],
            [pltpu.SemaphoreType.DMA for _ in range(NBUF)],
            [pltpu.SemaphoreType.DMA for _ in range(NBUF)],
        ],
        compiler_params=pltpu.CompilerParams(
            use_tc_tiling_on_sc=True, needs_layout_passes=False
        ),
    )
    def sc_kernel(
        xt_hbm, w_hbm, pe_hbm, out_hbm,
        idx_v, pe_v, rows, tbuf, idx2, poff, gsem, ssem,
    ):
        wid = lax.axis_index("s") * NC + lax.axis_index("c")
        pltpu.sync_copy(xt_hbm.at[:, wid], idx_v)
        pltpu.sync_copy(pe_hbm, pe_v)
        lane = lax.iota(jnp.int32, LANES)

        @pl.loop(0, ROUNDS)
        def _round(g):
            gdesc = []
            for k in range(NBUF):
                t = g * NBUF + k
                tt, r = lax.div(t, SUB), lax.rem(t, SUB)
                for q in range(gblk):
                    iv = idx_v[tt, r, pl.ds(q * LANES, LANES)]
                    idx2[k][pl.ds(q * LANES, LANES)] = lax.shift_right_logical(
                        iv, 1
                    )
                    poff[k][pl.ds(q * LANES, LANES)] = (iv & 1) * DIM
                gdesc.append(
                    pltpu.async_copy(w_hbm.at[idx2[k]], rows[k], gsem[k])
                )
            for k in range(NBUF):
                t = g * NBUF + k
                gdesc[k].wait()

                @pl.when(g > 0)
                def _():
                    pltpu.make_async_copy(
                        tbuf[k], out_hbm.at[0, :, 0], ssem[k]
                    ).wait()

                rowv = [lane + q * LANES for q in range(gblk)]
                colv = [poff[k][pl.ds(q * LANES, LANES)] for q in range(gblk)]
                pe_base = t * DIM

                @pl.loop(0, DIM)
                def _feat(j):
                    psplat = plsc.load_gather(
                        pe_v, [jnp.full((LANES,), pe_base + j, jnp.int32)]
                    )
                    m, r = lax.div(j, SUB), lax.rem(j, SUB)
                    for q in range(gblk):
                        vals = plsc.load_gather(rows[k], [rowv[q], colv[q] + j])
                        tbuf[k][m, r, pl.ds(q * LANES, LANES)] = vals + psplat

                pltpu.async_copy(tbuf[k], out_hbm.at[t, :, wid], ssem[k])

        for k in range(NBUF):
            pltpu.make_async_copy(tbuf[k], out_hbm.at[0, :, 0], ssem[k]).wait()

    return sc_kernel


def kernel(x, W):
    b, t = x.shape
    pe = _pe_table()
    bblk = b // NW
    xt = x.reshape(NW, bblk, t // SUB, SUB).transpose(2, 0, 3, 1)
    wv = W.reshape(W.shape[0] // 2, 2 * DIM)
    out5 = _make_sc_call(b)(xt, wv, pe.reshape(-1))
    return out5.transpose(2, 4, 0, 1, 3).reshape(b, t, DIM)
